# baseline (speedup 1.0000x reference)
import jax
import jax.numpy as jnp
from jax.experimental import pallas as pl
from jax.experimental.pallas import tpu as pltpu

_N = 2048
_D = 128
_OT_ITER = 20


def _gnncom_kernel(ft_ref, fs_ref, loss_ref, p_hbm, m_hbm,
                   k_ref, kt_ref, sem_m, sem_p):
    ft = ft_ref[...]
    fs = fs_ref[...]

    ftn = ft / jnp.maximum(
        jnp.sqrt(jnp.sum(ft * ft, axis=1, keepdims=True)), 1e-12)
    fsn = fs / jnp.maximum(
        jnp.sqrt(jnp.sum(fs * fs, axis=1, keepdims=True)), 1e-12)

    m = jax.lax.dot_general(
        ftn, fsn,
        dimension_numbers=(((1,), (1,)), ((), ())),
        preferred_element_type=jnp.float32)

    lo = jnp.min(m)
    inv = 1.0 / (jnp.max(m) - lo)
    k_ref[...] = (m - lo) * inv
    copy_m = pltpu.make_async_copy(k_ref, m_hbm, sem_m)
    copy_m.start()

    mt = jax.lax.dot_general(
        fsn, ftn,
        dimension_numbers=(((1,), (1,)), ((), ())),
        preferred_element_type=jnp.float32)
    kt_ref[...] = jnp.exp((mt - lo) * inv)

    copy_m.wait()
    k_ref[...] = jnp.exp(k_ref[...])

    r = 1.0 / _N
    c = 1.0 / _N
    v0 = jnp.ones((1, _N), dtype=jnp.float32)

    def body(_, v):
        kv = jax.lax.dot_general(
            v, kt_ref[...], dimension_numbers=(((1,), (0,)), ((), ())),
            preferred_element_type=jnp.float32)
        u = r / kv
        ktu = jax.lax.dot_general(
            u, k_ref[...], dimension_numbers=(((1,), (0,)), ((), ())),
            preferred_element_type=jnp.float32)
        return c / ktu

    v = jax.lax.fori_loop(0, _OT_ITER, body, v0)

    kv = jax.lax.dot_general(
        v, kt_ref[...], dimension_numbers=(((1,), (0,)), ((), ())),
        preferred_element_type=jnp.float32)
    u = 1.0 / kv
    ktu = jax.lax.dot_general(
        u, k_ref[...], dimension_numbers=(((1,), (0,)), ((), ())),
        preferred_element_type=jnp.float32)
    v = 1.0 / ktu

    ucol = u.reshape(_N, 1)
    p = ucol * k_ref[...] * v
    kt_ref[...] = p
    copy_p = pltpu.make_async_copy(kt_ref, p_hbm, sem_p)
    copy_p.start()

    row_i = jax.lax.broadcasted_iota(jnp.int32, (_N, _N), 0)
    col_i = jax.lax.broadcasted_iota(jnp.int32, (_N, _N), 1)
    terms = p * p - jnp.where(row_i == col_i, 2.0 * p, 0.0)
    loss_ref[...] = jnp.sqrt(
        jnp.sum(terms, keepdims=True) + jnp.float32(_N))

    copy_p.wait()


def kernel(ft, fs):
    loss2d, p, m = pl.pallas_call(
        _gnncom_kernel,
        out_shape=[
            jax.ShapeDtypeStruct((1, 1), jnp.float32),
            jax.ShapeDtypeStruct((_N, _N), jnp.float32),
            jax.ShapeDtypeStruct((_N, _N), jnp.float32),
        ],
        out_specs=[
            pl.BlockSpec(memory_space=pltpu.MemorySpace.VMEM),
            pl.BlockSpec(memory_space=pltpu.MemorySpace.HBM),
            pl.BlockSpec(memory_space=pltpu.MemorySpace.HBM),
        ],
        scratch_shapes=[
            pltpu.VMEM((_N, _N), jnp.float32),
            pltpu.VMEM((_N, _N), jnp.float32),
            pltpu.SemaphoreType.DMA,
            pltpu.SemaphoreType.DMA,
        ],
        compiler_params=pltpu.CompilerParams(
            vmem_limit_bytes=62 * 1024 * 1024),
    )(ft, fs)
    return (loss2d[0, 0], p, m)
